# baseline (speedup 1.0000x reference)
import jax
import jax.numpy as jnp
import numpy as np
from jax import lax
from jax.experimental import pallas as pl
from jax.experimental.pallas import tpu as pltpu
from jax.experimental.pallas import tpu_sc as plsc

KTOP = 128
L = 16
NC = 2
NS = 16
NWORK = NC * NS
UNROLL = 32
NHIST = 2

MIN32 = np.int32(-2**31)


def _to_ukey(v):
    u = plsc.bitcast(v, jnp.int32)
    return jnp.where(u < 0, ~u, u ^ MIN32)


def _from_ukey(k):
    u = jnp.where(k < 0, k ^ MIN32, ~k)
    return plsc.bitcast(u, jnp.float32)


def _body(x_hbm, out_hbm, xbuf, hist, obuf):
    n = x_hbm.shape[1]
    groups_per_batch = x_hbm.shape[2] // L
    gpw = (x_hbm.shape[0] * groups_per_batch) // NWORK

    wid = lax.axis_index("s") * NC + lax.axis_index("c")
    lanes = lax.iota(jnp.int32, L)
    ones = jnp.ones((L,), jnp.int32)
    zeros = jnp.zeros((L,), jnp.int32)

    @plsc.parallel_loop(0, NHIST * 256, unroll=8)
    def _zero0(j):
        hist[j] = zeros

    def scan_hist(r):
        @plsc.parallel_loop(0, 256, unroll=8,
                            carry=(zeros, zeros, zeros, zeros, zeros))
        def scan(i, carry):
            acc, bsel, before, hcross, found = carry
            j = 255 - i
            h = hist[j]
            hist[j] = zeros
            for hh in range(1, NHIST):
                h = h + hist[hh * 256 + j]
                hist[hh * 256 + j] = zeros
            acc_new = acc + h
            crossed = (found == 0) & (acc_new >= r)
            jv = jnp.full((L,), 0, jnp.int32) + j
            bsel = jnp.where(crossed, jv, bsel)
            before = jnp.where(crossed, acc, before)
            hcross = jnp.where(crossed, h, hcross)
            found = found | crossed.astype(jnp.int32)
            return acc_new, bsel, before, hcross, found

        _, bsel, before, hcross, _ = scan
        return bsel, before, hcross

    def group_body(t, _):
        gid = wid * gpw + t
        b = gid // groups_per_batch
        c0 = (gid - b * groups_per_batch) * L

        pltpu.sync_copy(x_hbm.at[b, :, pl.ds(c0, L)], xbuf)

        @plsc.parallel_loop(0, n, unroll=UNROLL)
        def _pass1(i):
            key = _to_ukey(xbuf[i])
            xbuf[i] = plsc.bitcast(key, jnp.float32)
            bucket = lax.shift_right_logical(key, 24)
            plsc.addupdate_scatter(
                hist, [(i & (NHIST - 1)) * 256 + bucket, lanes], ones)

        r = zeros + KTOP
        bsel, before, hcross = scan_hist(r)
        r = r - before
        prefix = bsel

        for level in (2, 3, 4):
            sh = 32 - 8 * level

            @plsc.parallel_loop(0, n, unroll=UNROLL)
            def _passl(i, sh=sh, prefix=prefix):
                key = plsc.bitcast(xbuf[i], jnp.int32)
                part = lax.shift_right_logical(key, sh + 8) == prefix
                bucket = jnp.bitwise_and(
                    lax.shift_right_logical(key, sh), jnp.int32(0xFF))
                plsc.addupdate_scatter(
                    hist, [(i & (NHIST - 1)) * 256 + bucket, lanes], ones,
                    mask=part)

            bsel, before, hcross = scan_hist(r)
            r = r - before
            prefix = jnp.bitwise_or(lax.shift_left(prefix, 8), bsel)

        thresh = prefix
        thresh_s = thresh ^ MIN32
        e_skip = hcross - r

        @plsc.parallel_loop(0, n, unroll=UNROLL, carry=(zeros, zeros))
        def compact(i, carry):
            cnt, eqseen = carry
            key = plsc.bitcast(xbuf[i], jnp.int32)
            gt = (key ^ MIN32) > thresh_s
            eq = key == thresh
            sel = gt | (eq & (eqseen >= e_skip))
            val = _from_ukey(key)
            plsc.store_scatter(obuf, [cnt, lanes], val, mask=sel)
            return cnt + sel.astype(jnp.int32), eqseen + eq.astype(jnp.int32)

        pltpu.sync_copy(obuf, out_hbm.at[b, :, pl.ds(c0, L)])
        return 0

    lax.fori_loop(0, gpw, group_body, 0)


@jax.jit
def kernel(x):
    batch, n, c = x.shape
    mesh = plsc.VectorSubcoreMesh(
        core_axis_name="c", subcore_axis_name="s", num_cores=NC, num_subcores=NS)
    run = pl.kernel(
        _body,
        out_type=jax.ShapeDtypeStruct((batch, KTOP, c), jnp.float32),
        mesh=mesh,
        compiler_params=pltpu.CompilerParams(
            use_tc_tiling_on_sc=False, needs_layout_passes=False),
        scratch_types=[
            pltpu.VMEM((n, L), jnp.float32),
            pltpu.VMEM((NHIST * 256, L), jnp.int32),
            pltpu.VMEM((KTOP, L), jnp.float32),
        ],
    )
    return run(x)


# speedup vs baseline: 1.1685x; 1.1685x over previous
import jax
import jax.numpy as jnp
import numpy as np
from jax import lax
from jax.experimental import pallas as pl
from jax.experimental.pallas import tpu as pltpu
from jax.experimental.pallas import tpu_sc as plsc

KTOP = 128
L = 16
NC = 2
NS = 16
NWORK = NC * NS
UNROLL = 16
NHIST = 2

MIN32 = np.int32(-2**31)


def _to_ukey(v):
    u = plsc.bitcast(v, jnp.int32)
    return jnp.where(u < 0, ~u, u ^ MIN32)


def _from_ukey(k):
    u = jnp.where(k < 0, k ^ MIN32, ~k)
    return plsc.bitcast(u, jnp.float32)


def _body(x_hbm, out_hbm, xbuf, hist, obuf):
    n = x_hbm.shape[1]
    groups_per_batch = x_hbm.shape[2] // L
    gpw = (x_hbm.shape[0] * groups_per_batch) // NWORK

    wid = lax.axis_index("s") * NC + lax.axis_index("c")
    lanes = lax.iota(jnp.int32, L)
    ones = jnp.ones((L,), jnp.int32)
    zeros = jnp.zeros((L,), jnp.int32)

    @plsc.parallel_loop(0, NHIST * 256, unroll=8)
    def _zero0(j):
        hist[j] = zeros

    def scan_hist(r):
        @plsc.parallel_loop(0, 256, unroll=4,
                            carry=(zeros, zeros, zeros, zeros, zeros))
        def scan(i, carry):
            acc, bsel, before, hcross, found = carry
            j = 255 - i
            h = hist[j]
            hist[j] = zeros
            for hh in range(1, NHIST):
                h = h + hist[hh * 256 + j]
                hist[hh * 256 + j] = zeros
            acc_new = acc + h
            crossed = (found == 0) & (acc_new >= r)
            jv = jnp.full((L,), 0, jnp.int32) + j
            bsel = jnp.where(crossed, jv, bsel)
            before = jnp.where(crossed, acc, before)
            hcross = jnp.where(crossed, h, hcross)
            found = found | crossed.astype(jnp.int32)
            return acc_new, bsel, before, hcross, found

        _, bsel, before, hcross, _ = scan
        return bsel, before, hcross

    def group_body(t, _):
        gid = wid * gpw + t
        b = gid // groups_per_batch
        c0 = (gid - b * groups_per_batch) * L

        pltpu.sync_copy(x_hbm.at[b, :, pl.ds(c0, L)], xbuf)

        @plsc.parallel_loop(0, n, unroll=UNROLL)
        def _pass1(i):
            key = _to_ukey(xbuf[i])
            xbuf[i] = plsc.bitcast(key, jnp.float32)
            bucket = lax.shift_right_logical(key, 24)
            plsc.addupdate_scatter(
                hist, [(i & (NHIST - 1)) * 256 + bucket, lanes], ones)

        r = zeros + KTOP
        bsel, before, hcross = scan_hist(r)
        r = r - before
        prefix = bsel

        for level in (2, 3, 4):
            sh = 32 - 8 * level

            @plsc.parallel_loop(0, n, unroll=UNROLL)
            def _passl(i, sh=sh, prefix=prefix):
                key = plsc.bitcast(xbuf[i], jnp.int32)
                part = lax.shift_right_logical(key, sh + 8) == prefix
                bucket = jnp.bitwise_and(
                    lax.shift_right_logical(key, sh), jnp.int32(0xFF))
                plsc.addupdate_scatter(
                    hist, [(i & (NHIST - 1)) * 256 + bucket, lanes], ones,
                    mask=part)

            bsel, before, hcross = scan_hist(r)
            r = r - before
            prefix = jnp.bitwise_or(lax.shift_left(prefix, 8), bsel)

        thresh = prefix
        thresh_s = thresh ^ MIN32
        e_skip = hcross - r

        @plsc.parallel_loop(0, n, unroll=UNROLL, carry=(zeros, zeros))
        def compact(i, carry):
            cnt, eqseen = carry
            key = plsc.bitcast(xbuf[i], jnp.int32)
            gt = (key ^ MIN32) > thresh_s
            eq = key == thresh
            sel = gt | (eq & (eqseen >= e_skip))
            val = _from_ukey(key)
            plsc.store_scatter(obuf, [cnt, lanes], val, mask=sel)
            return cnt + sel.astype(jnp.int32), eqseen + eq.astype(jnp.int32)

        pltpu.sync_copy(obuf, out_hbm.at[b, :, pl.ds(c0, L)])
        return 0

    lax.fori_loop(0, gpw, group_body, 0)


@jax.jit
def kernel(x):
    batch, n, c = x.shape
    mesh = plsc.VectorSubcoreMesh(
        core_axis_name="c", subcore_axis_name="s", num_cores=NC, num_subcores=NS)
    run = pl.kernel(
        _body,
        out_type=jax.ShapeDtypeStruct((batch, KTOP, c), jnp.float32),
        mesh=mesh,
        compiler_params=pltpu.CompilerParams(
            use_tc_tiling_on_sc=False, needs_layout_passes=False),
        scratch_types=[
            pltpu.VMEM((n, L), jnp.float32),
            pltpu.VMEM((NHIST * 256, L), jnp.int32),
            pltpu.VMEM((KTOP, L), jnp.float32),
        ],
    )
    return run(x)
